# baseline (speedup 1.0000x reference)
import functools

import jax
import jax.numpy as jnp
from jax import lax
from jax.experimental import pallas as pl
from jax.experimental.pallas import tpu as pltpu
from jax.experimental.pallas import tpu_sc as plsc

_HI = jax.lax.Precision.HIGHEST


def _top2_kernel(x_ref, cb_ref, i1_ref, i2_ref):
    x = x_ref[...]
    cb = cb_ref[...]
    bb = x.shape[0]
    k = cb.shape[0]

    cn = jnp.sum(cb * cb, axis=1, keepdims=True)
    cb_aug = jnp.concatenate([-2.0 * cb, cn], axis=1)
    x_aug = jnp.concatenate([x, jnp.ones((bb, 1), jnp.float32)], axis=1)
    d2 = jax.lax.dot_general(x_aug, cb_aug, (((1,), (1,)), ((), ())),
                             precision=_HI,
                             preferred_element_type=jnp.float32)

    iota = jax.lax.broadcasted_iota(jnp.int32, d2.shape, 1)
    m1 = jnp.min(d2, axis=1, keepdims=True)
    i1 = jnp.min(jnp.where(d2 == m1, iota, k), axis=1, keepdims=True)
    d2b = jnp.where(iota == i1, jnp.inf, d2)
    m2 = jnp.min(d2b, axis=1, keepdims=True)
    i2 = jnp.min(jnp.where(d2b == m2, iota, k), axis=1, keepdims=True)
    i1_ref[...] = i1[:, 0]
    i2_ref[...] = i2[:, 0]


def _refine_kernel(x_ref, c1_ref, c2_ref, i1_ref, i2_ref,
                   quant_ref, idx_ref, loss_ref):
    x = x_ref[...]
    d = x.shape[1]
    c1 = c1_ref[...][:, :d]
    c2 = c2_ref[...][:, :d]
    i1 = i1_ref[...][:, None]
    i2 = i2_ref[...][:, None]
    r1 = x - c1
    r2 = x - c2
    e1 = jnp.sum(r1 * r1, axis=1, keepdims=True)
    e2 = jnp.sum(r2 * r2, axis=1, keepdims=True)
    f1 = jnp.sqrt(e1)
    f2 = jnp.sqrt(e2)
    pick1 = (f1 < f2) | ((f1 == f2) & (i1 < i2))
    quant = jnp.where(pick1, c1, c2)
    diff = quant - x
    quant_ref[...] = x + diff
    idx_ref[...] = jnp.where(pick1, i1, i2)[:, 0]

    @pl.when(pl.program_id(0) == 0)
    def _init():
        loss_ref[...] = jnp.zeros((1, 1), jnp.float32)

    loss_ref[...] += jnp.sum(diff * diff, keepdims=True)


def _make_sc_gather(b, dp, n_workers, chunk):
    b_per_w = b // n_workers
    n_chunks = b_per_w // chunk
    mesh = plsc.VectorSubcoreMesh(core_axis_name="c", subcore_axis_name="s")

    @functools.partial(
        pl.kernel, mesh=mesh,
        out_type=[
            jax.ShapeDtypeStruct((b, dp), jnp.float32),
            jax.ShapeDtypeStruct((b, dp), jnp.float32),
        ],
        scratch_types=[
            pltpu.VMEM((chunk,), jnp.int32),
            pltpu.VMEM((chunk,), jnp.int32),
            pltpu.VMEM((chunk,), jnp.int32),
            pltpu.VMEM((chunk,), jnp.int32),
            pltpu.VMEM((chunk, dp), jnp.float32),
            pltpu.VMEM((chunk, dp), jnp.float32),
            pltpu.VMEM((chunk, dp), jnp.float32),
            pltpu.VMEM((chunk, dp), jnp.float32),
            pltpu.SemaphoreType.DMA,
            pltpu.SemaphoreType.DMA,
            pltpu.SemaphoreType.DMA,
            pltpu.SemaphoreType.DMA,
        ],
    )
    def sc_gather(cb_hbm, i1_hbm, i2_hbm, c1_hbm, c2_hbm,
                  ia0, ia1, ib0, ib1, ra0, ra1, rb0, rb1,
                  gs0, gs1, ws0, ws1):
        idx1 = (ia0, ia1)
        idx2 = (ib0, ib1)
        rows1 = (ra0, ra1)
        rows2 = (rb0, rb1)
        gsem = (gs0, gs1)
        wsem = (ws0, ws1)
        wid = lax.axis_index("s") * 2 + lax.axis_index("c")
        base = wid * b_per_w
        pending = [None, None]
        for cidx in range(n_chunks):
            p = cidx % 2
            off = base + cidx * chunk
            if pending[p] is not None:
                for w in pending[p]:
                    w.wait()
            pltpu.sync_copy(i1_hbm.at[pl.ds(off, chunk)], idx1[p])
            pltpu.sync_copy(i2_hbm.at[pl.ds(off, chunk)], idx2[p])
            g1 = pltpu.async_copy(cb_hbm.at[idx1[p]], rows1[p], gsem[p])
            g2 = pltpu.async_copy(cb_hbm.at[idx2[p]], rows2[p], gsem[p])
            g1.wait()
            g2.wait()
            w1 = pltpu.async_copy(rows1[p], c1_hbm.at[pl.ds(off, chunk)],
                                  wsem[p])
            w2 = pltpu.async_copy(rows2[p], c2_hbm.at[pl.ds(off, chunk)],
                                  wsem[p])
            pending[p] = (w1, w2)
        for pend in pending:
            if pend is not None:
                for w in pend:
                    w.wait()

    return sc_gather


def _top2(x_slice, codebook, bb):
    b, d = x_slice.shape
    k = codebook.shape[0]
    return pl.pallas_call(
        _top2_kernel,
        grid=(b // bb,),
        in_specs=[
            pl.BlockSpec((bb, d), lambda i: (i, 0)),
            pl.BlockSpec((k, d), lambda i: (0, 0)),
        ],
        out_specs=[
            pl.BlockSpec((bb,), lambda i: (i,)),
            pl.BlockSpec((bb,), lambda i: (i,)),
        ],
        out_shape=[
            jax.ShapeDtypeStruct((b,), jnp.int32),
            jax.ShapeDtypeStruct((b,), jnp.int32),
        ],
    )(x_slice, codebook)


def _refine(x_slice, c1, c2, i1, i2, bb2):
    b, d = x_slice.shape
    return pl.pallas_call(
        _refine_kernel,
        grid=(b // bb2,),
        in_specs=[
            pl.BlockSpec((bb2, d), lambda i: (i, 0)),
            pl.BlockSpec((bb2, 128), lambda i: (i, 0)),
            pl.BlockSpec((bb2, 128), lambda i: (i, 0)),
            pl.BlockSpec((bb2,), lambda i: (i,)),
            pl.BlockSpec((bb2,), lambda i: (i,)),
        ],
        out_specs=[
            pl.BlockSpec((bb2, d), lambda i: (i, 0)),
            pl.BlockSpec((bb2,), lambda i: (i,)),
            pl.BlockSpec((1, 1), lambda i: (0, 0)),
        ],
        out_shape=[
            jax.ShapeDtypeStruct((b, d), jnp.float32),
            jax.ShapeDtypeStruct((b,), jnp.int32),
            jax.ShapeDtypeStruct((1, 1), jnp.float32),
        ],
    )(x_slice, c1, c2, i1, i2)


def kernel(x, codebook):
    b, d = x.shape
    n_slices = 2
    bs = b // n_slices
    bb = 512
    bb2 = 2048
    dp = 128
    cb_pad = jnp.pad(codebook, ((0, 0), (0, dp - d)))
    sc_gather = _make_sc_gather(bs, dp, 32, 128)

    xs_l = [lax.slice_in_dim(x, s * bs, (s + 1) * bs, axis=0)
            for s in range(n_slices)]
    tops = [_top2(xs, codebook, bb) for xs in xs_l]
    rows = [sc_gather(cb_pad, i1, i2) for (i1, i2) in tops]
    quants, idxs, losses = [], [], []
    for xs, (i1, i2), (c1, c2) in zip(xs_l, tops, rows):
        quant_s, idx_s, loss_s = _refine(xs, c1, c2, i1, i2, bb2)
        quants.append(quant_s)
        idxs.append(idx_s)
        losses.append(loss_s[0, 0])

    quant = jnp.concatenate(quants, axis=0)
    idx = jnp.concatenate(idxs, axis=0)
    loss = sum(losses) / jnp.float32(b * d)
    return (quant, loss, loss, idx)


# speedup vs baseline: 1.1969x; 1.1969x over previous
import functools

import jax
import jax.numpy as jnp
from jax import lax
from jax.experimental import pallas as pl
from jax.experimental.pallas import tpu as pltpu
from jax.experimental.pallas import tpu_sc as plsc

_HI = jax.lax.Precision.HIGHEST


def _top2_kernel(x_ref, cb_ref, i1_ref, i2_ref):
    x = x_ref[...]
    rhs = cb_ref[...]
    bb = x.shape[0]
    k = rhs.shape[0]

    x_hi = x.astype(jnp.bfloat16)
    r1 = x - x_hi.astype(jnp.float32)
    x_mid = r1.astype(jnp.bfloat16)
    x_lo = (r1 - x_mid.astype(jnp.float32)).astype(jnp.bfloat16)
    ones = jnp.ones((bb, 3), jnp.bfloat16)
    lhs = jnp.concatenate(
        [x_hi, x_hi, x_mid, x_hi, x_lo, x_mid, ones], axis=1)
    d2 = jax.lax.dot_general(lhs, rhs, (((1,), (1,)), ((), ())),
                             preferred_element_type=jnp.float32)

    iota = jax.lax.broadcasted_iota(jnp.int32, d2.shape, 1)
    m1 = jnp.min(d2, axis=1, keepdims=True)
    i1 = jnp.min(jnp.where(d2 == m1, iota, k), axis=1, keepdims=True)
    d2b = jnp.where(iota == i1, jnp.inf, d2)
    m2 = jnp.min(d2b, axis=1, keepdims=True)
    i2 = jnp.min(jnp.where(d2b == m2, iota, k), axis=1, keepdims=True)
    i1_ref[...] = i1[:, 0]
    i2_ref[...] = i2[:, 0]


def _refine_kernel(x_ref, c1_ref, c2_ref, i1_ref, i2_ref,
                   quant_ref, idx_ref, loss_ref):
    x = x_ref[...]
    d = x.shape[1]
    c1 = c1_ref[...][:, :d]
    c2 = c2_ref[...][:, :d]
    i1 = i1_ref[...][:, None]
    i2 = i2_ref[...][:, None]
    r1 = x - c1
    r2 = x - c2
    e1 = jnp.sum(r1 * r1, axis=1, keepdims=True)
    e2 = jnp.sum(r2 * r2, axis=1, keepdims=True)
    f1 = jnp.sqrt(e1)
    f2 = jnp.sqrt(e2)
    pick1 = (f1 < f2) | ((f1 == f2) & (i1 < i2))
    quant = jnp.where(pick1, c1, c2)
    diff = quant - x
    quant_ref[...] = x + diff
    idx_ref[...] = jnp.where(pick1, i1, i2)[:, 0]

    @pl.when(pl.program_id(0) == 0)
    def _init():
        loss_ref[...] = jnp.zeros((1, 1), jnp.float32)

    loss_ref[...] += jnp.sum(diff * diff, keepdims=True)


def _make_sc_gather(b, dp, n_workers, chunk):
    b_per_w = b // n_workers
    n_chunks = b_per_w // chunk
    mesh = plsc.VectorSubcoreMesh(core_axis_name="c", subcore_axis_name="s")

    @functools.partial(
        pl.kernel, mesh=mesh,
        out_type=[
            jax.ShapeDtypeStruct((b, dp), jnp.float32),
            jax.ShapeDtypeStruct((b, dp), jnp.float32),
        ],
        scratch_types=[
            pltpu.VMEM((chunk,), jnp.int32),
            pltpu.VMEM((chunk,), jnp.int32),
            pltpu.VMEM((chunk,), jnp.int32),
            pltpu.VMEM((chunk,), jnp.int32),
            pltpu.VMEM((chunk, dp), jnp.float32),
            pltpu.VMEM((chunk, dp), jnp.float32),
            pltpu.VMEM((chunk, dp), jnp.float32),
            pltpu.VMEM((chunk, dp), jnp.float32),
            pltpu.SemaphoreType.DMA,
            pltpu.SemaphoreType.DMA,
            pltpu.SemaphoreType.DMA,
            pltpu.SemaphoreType.DMA,
        ],
    )
    def sc_gather(cb_hbm, i1_hbm, i2_hbm, c1_hbm, c2_hbm,
                  ia0, ia1, ib0, ib1, ra0, ra1, rb0, rb1,
                  gs0, gs1, ws0, ws1):
        idx1 = (ia0, ia1)
        idx2 = (ib0, ib1)
        rows1 = (ra0, ra1)
        rows2 = (rb0, rb1)
        gsem = (gs0, gs1)
        wsem = (ws0, ws1)
        wid = lax.axis_index("s") * 2 + lax.axis_index("c")
        base = wid * b_per_w
        pending = [None, None]
        for cidx in range(n_chunks):
            p = cidx % 2
            off = base + cidx * chunk
            if pending[p] is not None:
                for w in pending[p]:
                    w.wait()
            pltpu.sync_copy(i1_hbm.at[pl.ds(off, chunk)], idx1[p])
            pltpu.sync_copy(i2_hbm.at[pl.ds(off, chunk)], idx2[p])
            g1 = pltpu.async_copy(cb_hbm.at[idx1[p]], rows1[p], gsem[p])
            g2 = pltpu.async_copy(cb_hbm.at[idx2[p]], rows2[p], gsem[p])
            g1.wait()
            g2.wait()
            w1 = pltpu.async_copy(rows1[p], c1_hbm.at[pl.ds(off, chunk)],
                                  wsem[p])
            w2 = pltpu.async_copy(rows2[p], c2_hbm.at[pl.ds(off, chunk)],
                                  wsem[p])
            pending[p] = (w1, w2)
        for pend in pending:
            if pend is not None:
                for w in pend:
                    w.wait()

    return sc_gather


def _split3(a):
    hi = a.astype(jnp.bfloat16)
    r = a - hi.astype(jnp.float32)
    mid = r.astype(jnp.bfloat16)
    lo = (r - mid.astype(jnp.float32)).astype(jnp.bfloat16)
    return hi, mid, lo


def _build_rhs(codebook):
    cn = jnp.sum(codebook * codebook, axis=1, keepdims=True)
    c2 = -2.0 * codebook
    c2h, c2m, c2l = _split3(c2)
    cnh, cnm, cnl = _split3(cn)
    return jnp.concatenate(
        [c2h, c2m, c2h, c2l, c2h, c2m, cnh, cnm, cnl], axis=1)


def _top2(x_slice, rhs, bb):
    b, d = x_slice.shape
    k = rhs.shape[0]
    da = rhs.shape[1]
    return pl.pallas_call(
        _top2_kernel,
        grid=(b // bb,),
        in_specs=[
            pl.BlockSpec((bb, d), lambda i: (i, 0)),
            pl.BlockSpec((k, da), lambda i: (0, 0)),
        ],
        out_specs=[
            pl.BlockSpec((bb,), lambda i: (i,)),
            pl.BlockSpec((bb,), lambda i: (i,)),
        ],
        out_shape=[
            jax.ShapeDtypeStruct((b,), jnp.int32),
            jax.ShapeDtypeStruct((b,), jnp.int32),
        ],
    )(x_slice, rhs)


def _refine(x_slice, c1, c2, i1, i2, bb2):
    b, d = x_slice.shape
    return pl.pallas_call(
        _refine_kernel,
        grid=(b // bb2,),
        in_specs=[
            pl.BlockSpec((bb2, d), lambda i: (i, 0)),
            pl.BlockSpec((bb2, 128), lambda i: (i, 0)),
            pl.BlockSpec((bb2, 128), lambda i: (i, 0)),
            pl.BlockSpec((bb2,), lambda i: (i,)),
            pl.BlockSpec((bb2,), lambda i: (i,)),
        ],
        out_specs=[
            pl.BlockSpec((bb2, d), lambda i: (i, 0)),
            pl.BlockSpec((bb2,), lambda i: (i,)),
            pl.BlockSpec((1, 1), lambda i: (0, 0)),
        ],
        out_shape=[
            jax.ShapeDtypeStruct((b, d), jnp.float32),
            jax.ShapeDtypeStruct((b,), jnp.int32),
            jax.ShapeDtypeStruct((1, 1), jnp.float32),
        ],
    )(x_slice, c1, c2, i1, i2)


def kernel(x, codebook):
    b, d = x.shape
    n_slices = 2
    bs = b // n_slices
    bb = 1024
    bb2 = 2048
    dp = 128
    cb_pad = jnp.pad(codebook, ((0, 0), (0, dp - d)))
    sc_gather = _make_sc_gather(bs, dp, 32, 128)

    xs_l = [lax.slice_in_dim(x, s * bs, (s + 1) * bs, axis=0)
            for s in range(n_slices)]
    rhs = _build_rhs(codebook)
    tops = [_top2(xs, rhs, bb) for xs in xs_l]
    rows = [sc_gather(cb_pad, i1, i2) for (i1, i2) in tops]
    quants, idxs, losses = [], [], []
    for xs, (i1, i2), (c1, c2) in zip(xs_l, tops, rows):
        quant_s, idx_s, loss_s = _refine(xs, c1, c2, i1, i2, bb2)
        quants.append(quant_s)
        idxs.append(idx_s)
        losses.append(loss_s[0, 0])

    quant = jnp.concatenate(quants, axis=0)
    idx = jnp.concatenate(idxs, axis=0)
    loss = sum(losses) / jnp.float32(b * d)
    return (quant, loss, loss, idx)
